# baseline (speedup 1.0000x reference)
import jax
import jax.numpy as jnp
from jax.experimental import pallas as pl

_TN = 512


def _chamfer_tile_kernel(xa_ref, ya_ref, dx_ref, dy_ref):
    i = pl.program_id(1)
    xa = xa_ref[0]
    ya = ya_ref[0]
    d = jax.lax.dot_general(
        xa, ya, (((0,), (0,)), ((), ())),
        preferred_element_type=jnp.float32,
    )
    dx_ref[0, 0, pl.ds(i * _TN, _TN)] = jnp.min(d, axis=1)
    col_min = jnp.min(d, axis=0)

    @pl.when(i == 0)
    def _init():
        dy_ref[0, 0, :] = col_min

    @pl.when(i > 0)
    def _acc():
        dy_ref[0, 0, :] = jnp.minimum(dy_ref[0, 0, :], col_min)


@jax.jit
def kernel(x, y):
    B, C, N = x.shape
    M = y.shape[2]
    x2 = jnp.sum(x * x, axis=1)[:, None, :]
    y2 = jnp.sum(y * y, axis=1)[:, None, :]
    x2h = x2.astype(jnp.bfloat16)
    x2l = (x2 - x2h.astype(jnp.float32)).astype(jnp.bfloat16)
    y2h = y2.astype(jnp.bfloat16)
    y2l = (y2 - y2h.astype(jnp.float32)).astype(jnp.bfloat16)
    ones_n = jnp.ones((B, 1, N), jnp.bfloat16)
    ones_m = jnp.ones((B, 1, M), jnp.bfloat16)
    xa = jnp.concatenate(
        [(-2.0 * x).astype(jnp.bfloat16), ones_n, ones_n, x2h, x2l], axis=1)
    ya = jnp.concatenate(
        [y.astype(jnp.bfloat16), y2h, y2l, ones_m, ones_m], axis=1)
    CA = C + 4
    grid = (B, N // _TN)
    dx, dy = pl.pallas_call(
        _chamfer_tile_kernel,
        grid=grid,
        in_specs=[
            pl.BlockSpec((1, CA, _TN), lambda b, i: (b, 0, i)),
            pl.BlockSpec((1, CA, M), lambda b, i: (b, 0, 0)),
        ],
        out_specs=[
            pl.BlockSpec((1, 1, N), lambda b, i: (b, 0, 0)),
            pl.BlockSpec((1, 1, M), lambda b, i: (b, 0, 0)),
        ],
        out_shape=[
            jax.ShapeDtypeStruct((B, 1, N), jnp.float32),
            jax.ShapeDtypeStruct((B, 1, M), jnp.float32),
        ],
    )(xa, ya)
    dx = jnp.maximum(dx, 0.0)
    dy = jnp.maximum(dy, 0.0)
    return jnp.mean(dx) + jnp.mean(dy) * 10.0
